# Initial kernel scaffold; baseline (speedup 1.0000x reference)
import jax
import jax.numpy as jnp
from jax.experimental import pallas as pl


def kernel(x_author, x_paper, ei_writes, ei_written_by, w1s_wr, w1d_wr, a1s_wr, a1d_wr, b1_wr, w1s_wb, w1d_wb, a1s_wb, a1d_wb, b1_wb, w2_wr, a2s_wr, a2d_wr, b2_wr, w2_wb, a2s_wb, a2d_wb, b2_wb, lin_w, lin_b):
    raise NotImplementedError("write your pallas kernel here")


# speedup vs baseline: 1.1282x; 1.1282x over previous
import functools

import jax
import jax.numpy as jnp
from jax.experimental import pallas as pl

N_A = 10000
N_P = 10000
E = 160000
D_IN = 256
HID = 64
HEADS = 8
OUT = 256


def _cdiv(a, b):
    return (a + b - 1) // b


def _mm(x, w, bm=1024):
    M, K = x.shape
    K2, N = w.shape
    assert K == K2

    def body(x_ref, w_ref, o_ref):
        o_ref[...] = jnp.dot(x_ref[...], w_ref[...],
                             preferred_element_type=jnp.float32)

    return pl.pallas_call(
        body,
        grid=(_cdiv(M, bm),),
        in_specs=[pl.BlockSpec((bm, K), lambda i: (i, 0)),
                  pl.BlockSpec((K, N), lambda i: (0, 0))],
        out_specs=pl.BlockSpec((bm, N), lambda i: (i, 0)),
        out_shape=jax.ShapeDtypeStruct((M, N), jnp.float32),
    )(x, w)


def _gat_edge_jnp(hs, als, ald, src, dst, n_dst, H, C):
    e = jax.nn.leaky_relu(als[src] + ald[dst], negative_slope=0.2)
    ex = jnp.exp(e)
    den = jax.ops.segment_sum(ex, dst, num_segments=n_dst)
    num = jax.ops.segment_sum(
        hs.reshape(-1, H, C)[src] * ex[:, :, None], dst, num_segments=n_dst)
    return num.reshape(n_dst, H * C), den


def _gat(x_src, x_dst, ei, Ws, Wd, a_s, a_d, b, H, C, n_dst):
    hs = _mm(x_src, Ws)
    als = _mm(hs, _blockdiag(a_s, H, C), bm=2048)
    ald = _mm(x_dst, Wd @ _blockdiag(a_d, H, C), bm=2048)
    num, den = _gat_edge_jnp(hs, als, ald, ei[0], ei[1], n_dst, H, C)
    out = num / (jnp.repeat(den, C, axis=1) + 1e-16) + b
    return out


def _blockdiag(a, H, C):
    eye = jnp.eye(H, dtype=a.dtype)
    return (a[:, :, None] * eye[:, None, :]).reshape(H * C, H)


def kernel(x_author, x_paper, ei_writes, ei_written_by,
           w1s_wr, w1d_wr, a1s_wr, a1d_wr, b1_wr,
           w1s_wb, w1d_wb, a1s_wb, a1d_wb, b1_wb,
           w2_wr, a2s_wr, a2d_wr, b2_wr,
           w2_wb, a2s_wb, a2d_wb, b2_wb,
           lin_w, lin_b):
    p1 = _gat(x_author, x_paper, ei_writes, w1s_wr, w1d_wr, a1s_wr, a1d_wr,
              b1_wr, HEADS, HID, N_P)
    a1 = _gat(x_paper, x_author, ei_written_by, w1s_wb, w1d_wb, a1s_wb,
              a1d_wb, b1_wb, HEADS, HID, N_A)
    p2 = _gat(a1, p1, ei_writes, w2_wr, w2_wr, a2s_wr, a2d_wr, b2_wr,
              1, HID, N_P)
    a2 = _gat(p1, a1, ei_written_by, w2_wb, w2_wb, a2s_wb, a2d_wb, b2_wb,
              1, HID, N_A)
    out_author = _mm(a2, lin_w, bm=2048) + lin_b
    out_paper = _mm(p2, lin_w, bm=2048) + lin_b
    return (out_author, out_paper)


# speedup vs baseline: 1.1283x; 1.0001x over previous
import jax
import jax.numpy as jnp
from jax.experimental import pallas as pl

N_A = 10000
N_P = 10000
E = 160000
D_IN = 256
HID = 64
HEADS = 8
OUT = 256


def _cdiv(a, b):
    return (a + b - 1) // b


def _mm(x, w, bm=1024):
    M, K = x.shape
    K2, N = w.shape

    def body(x_ref, w_ref, o_ref):
        o_ref[...] = jnp.dot(x_ref[...], w_ref[...],
                             preferred_element_type=jnp.float32)

    return pl.pallas_call(
        body,
        grid=(_cdiv(M, bm),),
        in_specs=[pl.BlockSpec((bm, K), lambda i: (i, 0)),
                  pl.BlockSpec((K, N), lambda i: (0, 0))],
        out_specs=pl.BlockSpec((bm, N), lambda i: (i, 0)),
        out_shape=jax.ShapeDtypeStruct((M, N), jnp.float32),
    )(x, w)


def _gat_edge_jnp(hs, als, ald, src, dst, n_dst, H, C):
    e = jax.nn.leaky_relu(als[src] + ald[dst], negative_slope=0.2)
    ex = jnp.exp(e)
    den = jax.ops.segment_sum(ex, dst, num_segments=n_dst)
    num = jax.ops.segment_sum(
        hs.reshape(-1, H, C)[src] * ex[:, :, None], dst, num_segments=n_dst)
    return num.reshape(n_dst, H * C), den


def _blockdiag(a, H, C):
    eye = jnp.eye(H, dtype=a.dtype)
    return (a[:, :, None] * eye[:, None, :]).reshape(H * C, H)


def _gat(x_src, x_dst, ei, Ws, Wd, a_s, a_d, b, H, C, n_dst):
    hs = _mm(x_src, Ws)
    als = _mm(hs, _blockdiag(a_s, H, C), bm=2048)
    ald = _mm(x_dst, Wd @ _blockdiag(a_d, H, C), bm=2048)
    num, den = _gat_edge_jnp(hs, als, ald, ei[0], ei[1], n_dst, H, C)
    out = num / (jnp.repeat(den, C, axis=1) + 1e-16) + b
    return out


def kernel(x_author, x_paper, ei_writes, ei_written_by,
           w1s_wr, w1d_wr, a1s_wr, a1d_wr, b1_wr,
           w1s_wb, w1d_wb, a1s_wb, a1d_wb, b1_wb,
           w2_wr, a2s_wr, a2d_wr, b2_wr,
           w2_wb, a2s_wb, a2d_wb, b2_wb,
           lin_w, lin_b):
    p1 = _gat(x_author, x_paper, ei_writes, w1s_wr, w1d_wr, a1s_wr, a1d_wr,
              b1_wr, HEADS, HID, N_P)
    a1 = _gat(x_paper, x_author, ei_written_by, w1s_wb, w1d_wb, a1s_wb,
              a1d_wb, b1_wb, HEADS, HID, N_A)
    p2 = _gat(a1, p1, ei_writes, w2_wr, w2_wr, a2s_wr, a2d_wr, b2_wr,
              1, HID, N_P)
    a2 = _gat(p1, a1, ei_written_by, w2_wb, w2_wb, a2s_wb, a2d_wb, b2_wb,
              1, HID, N_A)
    out_author = _mm(a2, lin_w, bm=2048) + lin_b
    out_paper = _mm(p2, lin_w, bm=2048) + lin_b
    return (out_author, out_paper)
